# baseline (speedup 1.0000x reference)
import functools

import jax
import jax.numpy as jnp
from jax import lax
from jax.experimental import pallas as pl
from jax.experimental.pallas import tpu as pltpu
from jax.experimental.pallas import tpu_sc as plsc

_B = 4096
_S = 200
_D = 64
_V = 1000000
_L = 16
_NC, _NS = 2, 16
_NW = _NC * _NS
_BPW = _B // _NW
_CH = 16
_NCH = (_S + _CH - 1) // _CH
_REG = 264

_SW = 384
_NSLAB = _V // _SW
_VREM = _V - _NSLAB * _SW

_mesh = plsc.VectorSubcoreMesh(core_axis_name="c", subcore_axis_name="s")
_params_tiled = pltpu.CompilerParams(use_tc_tiling_on_sc=True,
                                     needs_layout_passes=False)
_params_linear = pltpu.CompilerParams(use_tc_tiling_on_sc=False,
                                      needs_layout_passes=False)


def _sc_relayout(tT, tail):
    @functools.partial(
        pl.kernel,
        out_type=jax.ShapeDtypeStruct((_V // 2, 2 * _D), jnp.float32),
        mesh=_mesh,
        compiler_params=_params_tiled,
        scratch_types=[
            pltpu.VMEM((2, _D, _SW), jnp.float32),
            pltpu.VMEM((2, _SW // 2, 2 * _D), jnp.float32),
            pltpu.SemaphoreType.DMA,
            pltpu.SemaphoreType.DMA,
        ],
    )
    def k(tT_hbm, tail_hbm, out_hbm, in_v, out_v, isem, osem):
        cid = lax.axis_index("c")
        sid = lax.axis_index("s")
        wid = sid * _NC + cid
        ns = 81 + jnp.where(wid < 12, 1, 0)
        lanes = jax.lax.broadcasted_iota(jnp.int32, (_L,), 0)
        rowc = lanes // 2
        colc = (lanes % 2) * _D

        def fire_in(p, s):
            pltpu.async_copy(
                tT_hbm.at[:, pl.ds(pl.multiple_of(s * _SW, _SW), _SW)],
                in_v.at[p], isem)

        def wait_in(p):
            pltpu.make_async_copy(tT_hbm.at[:, pl.ds(0, _SW)], in_v.at[p],
                                  isem).wait()

        def fire_out(p, s):
            pltpu.async_copy(
                out_v.at[p],
                out_hbm.at[pl.ds(pl.multiple_of(s * (_SW // 2), 8),
                                 _SW // 2)], osem)

        def drain_out(p, s):
            pltpu.make_async_copy(
                out_v.at[p],
                out_hbm.at[pl.ds(pl.multiple_of(s * (_SW // 2), 8),
                                 _SW // 2)], osem).wait()

        fire_in(0, wid)

        def slab_body(kk, carry):
            p = kk % 2
            s = wid + kk * _NW
            wait_in(p)
            @pl.when(kk + 1 < ns)
            def _():
                fire_in(p ^ 1, wid + (kk + 1) * _NW)
            @pl.when(kk >= 2)
            def _():
                drain_out(p, wid + (kk - 2) * _NW)

            def d_body(d, carry2):
                cvec = colc + d
                dvec = jnp.full((_L,), d, jnp.int32)
                for h in range(_SW // _L):
                    v16 = plsc.load_gather(in_v.at[p], [dvec, lanes + h * _L])
                    rvec = rowc + (h * _L // 2)
                    plsc.store_scatter(out_v.at[p], [rvec, cvec], v16)
                return carry2

            lax.fori_loop(0, _D, d_body, 0)
            fire_out(p, s)
            return carry

        lax.fori_loop(0, ns, slab_body, 0)

        @pl.when(ns >= 2)
        def _():
            drain_out((ns - 2) % 2, wid + (ns - 2) * _NW)
        drain_out((ns - 1) % 2, wid + (ns - 1) * _NW)

        @pl.when(wid == 12)
        def _():
            for q in range(2):
                pltpu.sync_copy(tail_hbm.at[:, pl.ds(q * _SW, _SW)],
                                in_v.at[0])

                def d_body(d, carry2, q=q):
                    cvec = colc + d
                    dvec = jnp.full((_L,), d, jnp.int32)
                    for h in range(_SW // _L):
                        v16 = plsc.load_gather(in_v.at[0],
                                               [dvec, lanes + h * _L])
                        rvec = rowc + (h * _L // 2)
                        plsc.store_scatter(out_v.at[0], [rvec, cvec], v16)
                    return carry2

                lax.fori_loop(0, _D, d_body, 0)
                pltpu.sync_copy(
                    out_v.at[0],
                    out_hbm.at[pl.ds((_V - 2 * _SW + q * _SW) // 2,
                                     _SW // 2)])

    return k(tT, tail)


def _sc_pool(bxT, lens, tpair):
    @functools.partial(
        pl.kernel,
        out_type=jax.ShapeDtypeStruct((2 * _B, 2 * _D), jnp.float32),
        mesh=_mesh,
        compiler_params=_params_linear,
        scratch_types=[
            pltpu.VMEM((_S, _BPW), jnp.int32),
            pltpu.VMEM((2, _NCH, _CH), jnp.int32),
            pltpu.VMEM((2, _NCH, _CH), jnp.int32),
            pltpu.VMEM((2, _NCH, _CH, 2 * _D), jnp.float32),
            pltpu.VMEM((_BPW,), jnp.int32),
            pltpu.VMEM((_CH, 2 * _D), jnp.float32),
            pltpu.VMEM_SHARED((_NS * _REG, 2 * _D), jnp.float32),
            pltpu.SemaphoreType.DMA,
            pltpu.SemaphoreType.DMA,
            pltpu.SemaphoreType.DMA,
        ],
    )
    def k(bx_hbm, lens_hbm, tab_hbm, out_hbm, blk_v, idx_v, sidx_v, rows_v,
          lens_v, zero_v, acc_sh, isem, gsem, ssem):
        cid = lax.axis_index("c")
        sid = lax.axis_index("s")
        wid = sid * _NC + cid
        base = pl.multiple_of(wid * _BPW, _BPW)
        reg0 = pl.multiple_of(sid * _REG, 8)

        pltpu.async_copy(bx_hbm.at[:, pl.ds(base, _BPW)], blk_v, isem)

        zeros16 = jnp.zeros((_L,), jnp.float32)
        for r in range(_CH):
            for j in range(2 * _D // _L):
                zero_v[r, pl.ds(j * _L, _L)] = zeros16
        for kk in range(_REG // _CH):
            pltpu.sync_copy(zero_v, acc_sh.at[pl.ds(reg0 + kk * _CH, _CH)])
        rem = _REG - (_REG // _CH) * _CH
        if rem:
            pltpu.sync_copy(zero_v.at[pl.ds(0, rem)],
                            acc_sh.at[pl.ds(reg0 + (_REG // _CH) * _CH, rem)])

        pltpu.sync_copy(lens_hbm.at[pl.ds(base, _BPW)], lens_v)
        pltpu.make_async_copy(bx_hbm.at[:, pl.ds(base, _BPW)], blk_v,
                              isem).wait()

        lanes = jax.lax.broadcasted_iota(jnp.int32, (_L,), 0)
        dump = reg0 + 2 * _BPW

        def n_chunks(l):
            return (l + _CH - 1) // _CH

        def fire_gathers(p, nc):
            def fg(c, _):
                pltpu.async_copy(tab_hbm.at[idx_v.at[p, c]],
                                 rows_v.at[p, c], gsem)
                return 0
            lax.fori_loop(0, nc, fg, 0)

        def wait_gathers(p, nc):
            def wg(c, _):
                pltpu.make_async_copy(tab_hbm.at[idx_v.at[p, 0]],
                                      rows_v.at[p, 0], gsem).wait()
                return 0
            lax.fori_loop(0, nc, wg, 0)

        def fire_scatters(p, nc):
            def fs(c, _):
                pltpu.async_copy(rows_v.at[p, c], acc_sh.at[sidx_v.at[p, c]],
                                 ssem, add=True)
                return 0
            lax.fori_loop(0, nc, fs, 0)

        def drain_scatters(p, nc):
            def ds_(c, _):
                pltpu.make_async_copy(rows_v.at[p, 0],
                                      acc_sh.at[sidx_v.at[p, 0]], ssem).wait()
                return 0
            lax.fori_loop(0, nc, ds_, 0)

        def group_body(g, carry):
            lv = lens_v[pl.ds(g * _L, _L)]
            for u in range(_L):
                p = u & 1
                i = g * _L + u
                l = lv[u]
                nc = n_chunks(l)
                if u >= 2:
                    drain_scatters(p, n_chunks(lv[u - 2]))
                slot0 = reg0 + 2 * i
                col = jnp.full((_L,), i, jnp.int32)
                for c in range(_NCH):
                    t = lanes + c * _CH
                    trow = jnp.minimum(t, _S - 1)
                    vals = plsc.load_gather(blk_v, [trow, col])
                    idx_v[p, c, pl.ds(0, _L)] = vals // 2
                    sidx_v[p, c, pl.ds(0, _L)] = jnp.where(
                        t < l, slot0 + (vals % 2), dump)
                fire_gathers(p, nc)
                wait_gathers(p, nc)
                fire_scatters(p, nc)
            drain_scatters(0, n_chunks(lv[_L - 2]))
            drain_scatters(1, n_chunks(lv[_L - 1]))
            return carry

        lax.fori_loop(0, _BPW // _L, group_body, 0)
        pltpu.sync_copy(acc_sh.at[pl.ds(reg0, 2 * _BPW)],
                        out_hbm.at[pl.ds(2 * base, 2 * _BPW)])

    return k(bxT, lens, tpair)


def _tc_head(doc2, lens2, W, b2):
    cat = W.shape[0]

    def body(x_ref, l_ref, w_ref, b_ref, o_ref):
        x = x_ref[...]
        doc_sum = x[:, 0:_D] + x[:, 3 * _D:4 * _D]
        denom = jnp.maximum(l_ref[...].astype(jnp.float32), 1.0)
        doc = doc_sum / denom
        z = lax.dot_general(doc, w_ref[...], (((1,), (1,)), ((), ())),
                            preferred_element_type=jnp.float32)
        z = z + b_ref[...]
        m = jnp.max(z, axis=-1, keepdims=True)
        e = jnp.exp(z - m)
        s = jnp.sum(e, axis=-1, keepdims=True)
        o_ref[...] = (z - m) - jnp.log(s)

    return pl.pallas_call(
        body,
        out_shape=jax.ShapeDtypeStruct((_B, cat), jnp.float32),
    )(doc2, lens2, W, b2)


def kernel(batch_x, batch_lens, emb_table, W, b):
    bx = batch_x.astype(jnp.int32)
    lens = batch_lens.astype(jnp.int32)
    tT = emb_table.T
    tpair = _sc_relayout(tT, tT[:, _V - 2 * _SW:])
    doc2 = _sc_pool(bx.T, lens, tpair)
    return _tc_head(doc2.reshape(_B, 4 * _D), lens.reshape(_B, 1), W,
                    b.reshape(1, -1))


# speedup vs baseline: 1.9924x; 1.9924x over previous
import functools

import jax
import jax.numpy as jnp
from jax import lax
from jax.experimental import pallas as pl
from jax.experimental.pallas import tpu as pltpu
from jax.experimental.pallas import tpu_sc as plsc

_B = 4096
_S = 200
_D = 64
_L = 16
_NC, _NS = 2, 16
_NW = _NC * _NS
_BPW = _B // _NW
_CH = 32
_NCH = (_S + _CH - 1) // _CH
_SP = _NCH * _CH
_REG = 136


def _sc_pool(bxT, lens, table):
    mesh = plsc.VectorSubcoreMesh(core_axis_name="c", subcore_axis_name="s")

    @functools.partial(
        pl.kernel,
        out_type=jax.ShapeDtypeStruct((_B, _D), jnp.float32),
        mesh=mesh,
        compiler_params=pltpu.CompilerParams(use_tc_tiling_on_sc=False,
                                             needs_layout_passes=False),
        scratch_types=[
            pltpu.VMEM((_S, _BPW), jnp.int32),
            pltpu.VMEM((3, _NCH, _CH), jnp.int32),
            pltpu.VMEM((3, _NCH, _CH), jnp.int32),
            pltpu.VMEM((3, _NCH, _CH, _D), jnp.float32),
            pltpu.VMEM((_BPW,), jnp.int32),
            pltpu.VMEM((_CH, _D), jnp.float32),
            pltpu.VMEM_SHARED((_NS * _REG, _D), jnp.float32),
            pltpu.SemaphoreType.DMA,
            pltpu.SemaphoreType.DMA,
            pltpu.SemaphoreType.DMA,
        ],
    )
    def k(bx_hbm, lens_hbm, table_hbm, out_hbm, blk_v, idx_v, sidx_v, rows_v,
          lens_v, zero_v, acc_sh, isem, gsem, ssem):
        cid = lax.axis_index("c")
        sid = lax.axis_index("s")
        wid = sid * _NC + cid
        base = wid * _BPW
        reg0 = sid * _REG

        pltpu.async_copy(bx_hbm.at[:, pl.ds(base, _BPW)], blk_v, isem)

        zeros16 = jnp.zeros((_L,), jnp.float32)
        for r in range(_CH):
            for j in range(4):
                zero_v[r, pl.ds(j * _L, _L)] = zeros16
        for kk in range(4):
            pltpu.sync_copy(zero_v, acc_sh.at[pl.ds(reg0 + kk * _CH, _CH)])
        pltpu.sync_copy(zero_v.at[pl.ds(0, _REG - 4 * _CH)],
                        acc_sh.at[pl.ds(reg0 + 4 * _CH, _REG - 4 * _CH)])

        pltpu.sync_copy(lens_hbm.at[pl.ds(base, _BPW)], lens_v)
        pltpu.make_async_copy(bx_hbm.at[:, pl.ds(base, _BPW)], blk_v,
                              isem).wait()

        lanes = jax.lax.broadcasted_iota(jnp.int32, (_L,), 0)
        dump = reg0 + _BPW

        def n_chunks(l):
            return (l + _CH - 1) // _CH

        def fire_gathers(p, nc):
            def fg(c, _):
                pltpu.async_copy(table_hbm.at[idx_v.at[p, c]],
                                 rows_v.at[p, c], gsem)
                return 0
            lax.fori_loop(0, nc, fg, 0)

        def wait_gathers(p, nc):
            def wg(c, _):
                pltpu.make_async_copy(table_hbm.at[idx_v.at[p, 0]],
                                      rows_v.at[p, 0], gsem).wait()
                return 0
            lax.fori_loop(0, nc, wg, 0)

        def fire_scatters(p, nc):
            def fs(c, _):
                pltpu.async_copy(rows_v.at[p, c], acc_sh.at[sidx_v.at[p, c]],
                                 ssem, add=True)
                return 0
            lax.fori_loop(0, nc, fs, 0)

        def drain_scatters(p, nc):
            def ds_(c, _):
                pltpu.make_async_copy(rows_v.at[p, 0],
                                      acc_sh.at[sidx_v.at[p, 0]], ssem).wait()
                return 0
            lax.fori_loop(0, nc, ds_, 0)

        def build_item(p, i, l):
            slot = reg0 + i
            col = jnp.full((_L,), i, jnp.int32)
            for c in range(_NCH):
                for h in range(2):
                    t = lanes + (c * _CH + h * _L)
                    trow = jnp.minimum(t, _S - 1)
                    vals = plsc.load_gather(blk_v, [trow, col])
                    idx_v[p, c, pl.ds(h * _L, _L)] = vals
                    sidx_v[p, c, pl.ds(h * _L, _L)] = jnp.where(
                        t < l, slot, dump)

        def group_body(g, carry):
            lv = lens_v[pl.ds(g * _L, _L)]
            for u in range(_L):
                p = u % 3
                if u >= 2:
                    drain_scatters((u - 2) % 3, n_chunks(lv[u - 2]))
                build_item(p, g * _L + u, lv[u])
                fire_gathers(p, n_chunks(lv[u]))
                if u >= 1:
                    wait_gathers((u - 1) % 3, n_chunks(lv[u - 1]))
                    fire_scatters((u - 1) % 3, n_chunks(lv[u - 1]))
            wait_gathers((_L - 1) % 3, n_chunks(lv[_L - 1]))
            fire_scatters((_L - 1) % 3, n_chunks(lv[_L - 1]))
            drain_scatters((_L - 2) % 3, n_chunks(lv[_L - 2]))
            drain_scatters((_L - 1) % 3, n_chunks(lv[_L - 1]))
            return carry

        lax.fori_loop(0, _BPW // _L, group_body, 0)
        pltpu.sync_copy(acc_sh.at[pl.ds(reg0, _BPW)],
                        out_hbm.at[pl.ds(base, _BPW)])

    return k(bxT, lens, table)


def _tc_head(doc_sum, lens2, W, b2):
    cat = W.shape[0]

    def body(x_ref, l_ref, w_ref, b_ref, o_ref):
        x = x_ref[...]
        denom = jnp.maximum(l_ref[...].astype(jnp.float32), 1.0)
        doc = x / denom
        z = lax.dot_general(doc, w_ref[...], (((1,), (1,)), ((), ())),
                            preferred_element_type=jnp.float32)
        z = z + b_ref[...]
        m = jnp.max(z, axis=-1, keepdims=True)
        e = jnp.exp(z - m)
        s = jnp.sum(e, axis=-1, keepdims=True)
        o_ref[...] = (z - m) - jnp.log(s)

    return pl.pallas_call(
        body,
        out_shape=jax.ShapeDtypeStruct((_B, cat), jnp.float32),
    )(doc_sum, lens2, W, b2)


def kernel(batch_x, batch_lens, emb_table, W, b):
    bx = batch_x.astype(jnp.int32)
    lens = batch_lens.astype(jnp.int32)
    doc_sum = _sc_pool(bx.T, lens, emb_table)
    return _tc_head(doc_sum, lens.reshape(_B, 1), W, b.reshape(1, -1))


# speedup vs baseline: 1.9962x; 1.0019x over previous
import functools

import jax
import jax.numpy as jnp
from jax import lax
from jax.experimental import pallas as pl
from jax.experimental.pallas import tpu as pltpu
from jax.experimental.pallas import tpu_sc as plsc

_B = 4096
_S = 200
_D = 64
_L = 16
_NC, _NS = 2, 16
_NW = _NC * _NS
_BPW = _B // _NW
_CH = 32
_NCH = (_S + _CH - 1) // _CH
_SP = _NCH * _CH
_REG = 136


def _sc_pool(bxT, lens, table):
    mesh = plsc.VectorSubcoreMesh(core_axis_name="c", subcore_axis_name="s")

    @functools.partial(
        pl.kernel,
        out_type=jax.ShapeDtypeStruct((_B, _D), jnp.float32),
        mesh=mesh,
        compiler_params=pltpu.CompilerParams(use_tc_tiling_on_sc=False,
                                             needs_layout_passes=False),
        scratch_types=[
            pltpu.VMEM((_S, _BPW), jnp.int32),
            pltpu.VMEM((3, _NCH, _CH), jnp.int32),
            pltpu.VMEM((3, _NCH, _CH), jnp.int32),
            pltpu.VMEM((3, _NCH, _CH, _D), jnp.float32),
            pltpu.VMEM((_BPW,), jnp.int32),
            pltpu.VMEM((_CH, _D), jnp.float32),
            pltpu.VMEM_SHARED((_NS * _REG, _D), jnp.float32),
            pltpu.SemaphoreType.DMA,
            pltpu.SemaphoreType.DMA,
            pltpu.SemaphoreType.DMA,
        ],
    )
    def k(bx_hbm, lens_hbm, table_hbm, out_hbm, blk_v, idx_v, sidx_v, rows_v,
          lens_v, zero_v, acc_sh, isem, gsem, ssem):
        cid = lax.axis_index("c")
        sid = lax.axis_index("s")
        wid = sid * _NC + cid
        base = wid * _BPW
        reg0 = sid * _REG

        pltpu.async_copy(bx_hbm.at[:, pl.ds(base, _BPW)], blk_v, isem)

        zeros16 = jnp.zeros((_L,), jnp.float32)
        for r in range(_CH):
            for j in range(4):
                zero_v[r, pl.ds(j * _L, _L)] = zeros16
        for kk in range(4):
            pltpu.sync_copy(zero_v, acc_sh.at[pl.ds(reg0 + kk * _CH, _CH)])
        pltpu.sync_copy(zero_v.at[pl.ds(0, _REG - 4 * _CH)],
                        acc_sh.at[pl.ds(reg0 + 4 * _CH, _REG - 4 * _CH)])

        pltpu.sync_copy(lens_hbm.at[pl.ds(base, _BPW)], lens_v)
        pltpu.make_async_copy(bx_hbm.at[:, pl.ds(base, _BPW)], blk_v,
                              isem).wait()

        lanes = jax.lax.broadcasted_iota(jnp.int32, (_L,), 0)
        dump = reg0 + _BPW

        def n_chunks(l):
            return (l + _CH - 1) // _CH

        def fire_gathers(p, nc):
            def fg(c, _):
                pltpu.async_copy(table_hbm.at[idx_v.at[p, c]],
                                 rows_v.at[p, c], gsem)
                return 0
            lax.fori_loop(0, nc, fg, 0)

        def wait_gathers(p, nc):
            def wg(c, _):
                pltpu.make_async_copy(table_hbm.at[idx_v.at[p, 0]],
                                      rows_v.at[p, 0], gsem).wait()
                return 0
            lax.fori_loop(0, nc, wg, 0)

        def fire_scatters(p, nc):
            def fs(c, _):
                pltpu.async_copy(rows_v.at[p, c], acc_sh.at[sidx_v.at[p, c]],
                                 ssem, add=True)
                return 0
            lax.fori_loop(0, nc, fs, 0)

        def drain_scatters(p, nc):
            def ds_(c, _):
                pltpu.make_async_copy(rows_v.at[p, 0],
                                      acc_sh.at[sidx_v.at[p, 0]], ssem).wait()
                return 0
            lax.fori_loop(0, nc, ds_, 0)

        def build_item(p, i, l):
            slot = reg0 + i
            col = jnp.full((_L,), i, jnp.int32)
            for c in range(_NCH):
                for h in range(2):
                    t = lanes + (c * _CH + h * _L)
                    trow = jnp.minimum(t, _S - 1)
                    vals = plsc.load_gather(blk_v, [trow, col])
                    idx_v[p, c, pl.ds(h * _L, _L)] = vals
                    sidx_v[p, c, pl.ds(h * _L, _L)] = jnp.where(
                        t < l, slot, dump)

        def group_body(g, carry):
            ncA, ncB = carry
            lv = lens_v[pl.ds(g * _L, _L)]
            for u in range(_L):
                i = g * _L + u
                p = i % 3
                nc_u = n_chunks(lv[u])
                ncm2 = ncA if u == 0 else (ncB if u == 1
                                           else n_chunks(lv[u - 2]))
                drain_scatters((i + 1) % 3, ncm2)
                build_item(p, i, lv[u])
                fire_gathers(p, nc_u)
                ncm1 = ncB if u == 0 else n_chunks(lv[u - 1])
                wait_gathers((i + 2) % 3, ncm1)
                fire_scatters((i + 2) % 3, ncm1)
            return (n_chunks(lv[_L - 2]), n_chunks(lv[_L - 1]))

        zero = jnp.int32(0)
        ncA_f, ncB_f = lax.fori_loop(0, _BPW // _L, group_body,
                                     (zero, zero))
        p_last = (_BPW - 1) % 3
        wait_gathers(p_last, ncB_f)
        fire_scatters(p_last, ncB_f)
        drain_scatters((_BPW - 2) % 3, ncA_f)
        drain_scatters(p_last, ncB_f)
        pltpu.sync_copy(acc_sh.at[pl.ds(reg0, _BPW)],
                        out_hbm.at[pl.ds(base, _BPW)])

    return k(bxT, lens, table)


def _tc_head(doc_sum, lens2, W, b2):
    cat = W.shape[0]

    def body(x_ref, l_ref, w_ref, b_ref, o_ref):
        x = x_ref[...]
        denom = jnp.maximum(l_ref[...].astype(jnp.float32), 1.0)
        doc = x / denom
        z = lax.dot_general(doc, w_ref[...], (((1,), (1,)), ((), ())),
                            preferred_element_type=jnp.float32)
        z = z + b_ref[...]
        m = jnp.max(z, axis=-1, keepdims=True)
        e = jnp.exp(z - m)
        s = jnp.sum(e, axis=-1, keepdims=True)
        o_ref[...] = (z - m) - jnp.log(s)

    return pl.pallas_call(
        body,
        out_shape=jax.ShapeDtypeStruct((_B, cat), jnp.float32),
    )(doc_sum, lens2, W, b2)


def kernel(batch_x, batch_lens, emb_table, W, b):
    bx = batch_x.astype(jnp.int32)
    lens = batch_lens.astype(jnp.int32)
    doc_sum = _sc_pool(bx.T, lens, emb_table)
    return _tc_head(doc_sum, lens.reshape(_B, 1), W, b.reshape(1, -1))
